# initial kernel scaffold (unmeasured)
import jax
import jax.numpy as jnp
from jax import lax
from jax.experimental import pallas as pl
from jax.experimental.pallas import tpu as pltpu

N_DEV = 8
B = 2
SQ = 512
SKV = 512
H_LOC = 8
DH = 64
BLK = 64
D_MODEL = 768


def kernel(x, Wq, K_ext, V_ext, Wo):
    def body(x_ref, wq_ref, k_ref, v_ref, wo_ref, out_ref,
             k_buf, v_buf, q_buf, ctx_buf, comm,
             scat_send_sems, kv_recv_sems, ring_send_sems, ring_recv_sems,
             local_sems):
        my_i = lax.axis_index("i")
        right = lax.rem(my_i + 1, N_DEV)

        @pl.when(my_i == 0)
        def _():
            ck = pltpu.make_async_copy(
                k_ref.at[:, :, 0:H_LOC, :], k_buf, local_sems.at[0])
            cv = pltpu.make_async_copy(
                v_ref.at[:, :, 0:H_LOC, :], v_buf, local_sems.at[1])
            ck.start()
            cv.start()
            rdmas = []
            for d in range(1, N_DEV):
                rk = pltpu.make_async_remote_copy(
                    src_ref=k_ref.at[:, :, d * H_LOC:(d + 1) * H_LOC, :],
                    dst_ref=k_buf,
                    send_sem=scat_send_sems.at[d - 1, 0],
                    recv_sem=kv_recv_sems.at[0],
                    device_id=(d,),
                    device_id_type=pl.DeviceIdType.MESH,
                )
                rv = pltpu.make_async_remote_copy(
                    src_ref=v_ref.at[:, :, d * H_LOC:(d + 1) * H_LOC, :],
                    dst_ref=v_buf,
                    send_sem=scat_send_sems.at[d - 1, 1],
                    recv_sem=kv_recv_sems.at[1],
                    device_id=(d,),
                    device_id_type=pl.DeviceIdType.MESH,
                )
                rk.start()
                rv.start()
                rdmas.append((rk, rv))

        for b in range(B):
            q_buf[b, :, :] = jnp.dot(
                x_ref[b, :, :], wq_ref[:, :],
                preferred_element_type=jnp.float32)

        @pl.when(my_i == 0)
        def _():
            pltpu.make_async_copy(
                k_ref.at[:, :, 0:H_LOC, :], k_buf, local_sems.at[0]).wait()
            pltpu.make_async_copy(
                v_ref.at[:, :, 0:H_LOC, :], v_buf, local_sems.at[1]).wait()
            for d in range(1, N_DEV):
                pltpu.make_async_remote_copy(
                    src_ref=k_ref.at[:, :, d * H_LOC:(d + 1) * H_LOC, :],
                    dst_ref=k_buf,
                    send_sem=scat_send_sems.at[d - 1, 0],
                    recv_sem=kv_recv_sems.at[0],
                    device_id=(d,),
                    device_id_type=pl.DeviceIdType.MESH,
                ).wait_send()
                pltpu.make_async_remote_copy(
                    src_ref=v_ref.at[:, :, d * H_LOC:(d + 1) * H_LOC, :],
                    dst_ref=v_buf,
                    send_sem=scat_send_sems.at[d - 1, 1],
                    recv_sem=kv_recv_sems.at[1],
                    device_id=(d,),
                    device_id_type=pl.DeviceIdType.MESH,
                ).wait_send()

        @pl.when(my_i != 0)
        def _():
            pltpu.make_async_remote_copy(
                src_ref=k_buf, dst_ref=k_buf,
                send_sem=scat_send_sems.at[0, 0],
                recv_sem=kv_recv_sems.at[0],
                device_id=(0,),
                device_id_type=pl.DeviceIdType.MESH,
            ).wait_recv()
            pltpu.make_async_remote_copy(
                src_ref=v_buf, dst_ref=v_buf,
                send_sem=scat_send_sems.at[0, 1],
                recv_sem=kv_recv_sems.at[1],
                device_id=(0,),
                device_id_type=pl.DeviceIdType.MESH,
            ).wait_recv()

        qb = lax.broadcasted_iota(jnp.int32, (SQ, SKV), 0) // BLK
        kb = lax.broadcasted_iota(jnp.int32, (SQ, SKV), 1) // BLK
        mask = kb <= qb

        for b in range(B):
            for h in range(H_LOC):
                q = q_buf[b, :, h * DH:(h + 1) * DH]
                k = k_buf[b, :, h, :]
                s = lax.dot_general(
                    q, k, (((1,), (1,)), ((), ())),
                    preferred_element_type=jnp.float32) * 0.125
                s = jnp.where(mask, s, -1e9)
                m = jnp.max(s, axis=1, keepdims=True)
                w = jnp.exp(s - m)
                w = w / jnp.sum(w, axis=1, keepdims=True)
                ctx_buf[b, :, h * DH:(h + 1) * DH] = jnp.dot(
                    w, v_buf[b, :, h, :],
                    preferred_element_type=jnp.float32)

        for b in range(B):
            part = jnp.dot(ctx_buf[b, :, :], wo_ref[:, :],
                           preferred_element_type=jnp.float32)
            out_ref[b, :, :] = part
            comm[0, b, :, :] = part

        for h in range(N_DEV - 1):
            rdma = pltpu.make_async_remote_copy(
                src_ref=comm.at[h],
                dst_ref=comm.at[h + 1],
                send_sem=ring_send_sems.at[h],
                recv_sem=ring_recv_sems.at[h],
                device_id=(right,),
                device_id_type=pl.DeviceIdType.MESH,
            )
            rdma.start()
            rdma.wait()
            out_ref[:, :, :] += comm[h + 1]

    return pl.pallas_call(
        body,
        out_shape=jax.ShapeDtypeStruct((B, SQ, D_MODEL), jnp.float32),
        in_specs=[
            pl.BlockSpec(memory_space=pltpu.VMEM),
            pl.BlockSpec(memory_space=pltpu.VMEM),
            pl.BlockSpec(memory_space=pltpu.ANY),
            pl.BlockSpec(memory_space=pltpu.ANY),
            pl.BlockSpec(memory_space=pltpu.VMEM),
        ],
        out_specs=pl.BlockSpec(memory_space=pltpu.VMEM),
        scratch_shapes=[
            pltpu.VMEM((B, SKV, H_LOC, DH), jnp.float32),
            pltpu.VMEM((B, SKV, H_LOC, DH), jnp.float32),
            pltpu.VMEM((B, SQ, H_LOC * DH), jnp.float32),
            pltpu.VMEM((B, SQ, H_LOC * DH), jnp.float32),
            pltpu.VMEM((N_DEV, B, SQ, D_MODEL), jnp.float32),
            pltpu.SemaphoreType.DMA((N_DEV - 1, 2)),
            pltpu.SemaphoreType.DMA((2,)),
            pltpu.SemaphoreType.DMA((N_DEV - 1,)),
            pltpu.SemaphoreType.DMA((N_DEV - 1,)),
            pltpu.SemaphoreType.DMA((2,)),
        ],
    )(x, Wq, K_ext, V_ext, Wo)


# baseline (device time: 695822 ns/iter reference)
import jax
import jax.numpy as jnp
from jax import lax
from jax.experimental import pallas as pl
from jax.experimental.pallas import tpu as pltpu

N_DEV = 8
B = 2
SQ = 512
SKV = 512
H_LOC = 8
DH = 64
BLK = 64
D_MODEL = 768


def kernel(x, Wq, K_ext, V_ext, Wo):
    def body(x_ref, wq_ref, k_ref, v_ref, wo_ref, out_ref,
             k_buf, v_buf, q_buf, ctx_buf, comm,
             scat_send_sems, kv_recv_sems, ring_send_sems, ring_recv_sems,
             local_sems):
        my_i = lax.axis_index("i")
        right = lax.rem(my_i + 1, N_DEV)

        barrier_sem = pltpu.get_barrier_semaphore()
        for d in range(N_DEV):
            pl.semaphore_signal(
                barrier_sem, inc=1,
                device_id=(d,), device_id_type=pl.DeviceIdType.MESH)
        pl.semaphore_wait(barrier_sem, N_DEV)

        @pl.when(my_i == 0)
        def _():
            ck = pltpu.make_async_copy(
                k_ref.at[:, :, 0:H_LOC, :], k_buf, local_sems.at[0])
            cv = pltpu.make_async_copy(
                v_ref.at[:, :, 0:H_LOC, :], v_buf, local_sems.at[1])
            ck.start()
            cv.start()
            rdmas = []
            for d in range(1, N_DEV):
                rk = pltpu.make_async_remote_copy(
                    src_ref=k_ref.at[:, :, d * H_LOC:(d + 1) * H_LOC, :],
                    dst_ref=k_buf,
                    send_sem=scat_send_sems.at[d - 1, 0],
                    recv_sem=kv_recv_sems.at[0],
                    device_id=(d,),
                    device_id_type=pl.DeviceIdType.MESH,
                )
                rv = pltpu.make_async_remote_copy(
                    src_ref=v_ref.at[:, :, d * H_LOC:(d + 1) * H_LOC, :],
                    dst_ref=v_buf,
                    send_sem=scat_send_sems.at[d - 1, 1],
                    recv_sem=kv_recv_sems.at[1],
                    device_id=(d,),
                    device_id_type=pl.DeviceIdType.MESH,
                )
                rk.start()
                rv.start()
                rdmas.append((rk, rv))

        for b in range(B):
            q_buf[b, :, :] = jnp.dot(
                x_ref[b, :, :], wq_ref[:, :],
                preferred_element_type=jnp.float32)

        @pl.when(my_i == 0)
        def _():
            pltpu.make_async_copy(
                k_ref.at[:, :, 0:H_LOC, :], k_buf, local_sems.at[0]).wait()
            pltpu.make_async_copy(
                v_ref.at[:, :, 0:H_LOC, :], v_buf, local_sems.at[1]).wait()
            for d in range(1, N_DEV):
                pltpu.make_async_remote_copy(
                    src_ref=k_ref.at[:, :, d * H_LOC:(d + 1) * H_LOC, :],
                    dst_ref=k_buf,
                    send_sem=scat_send_sems.at[d - 1, 0],
                    recv_sem=kv_recv_sems.at[0],
                    device_id=(d,),
                    device_id_type=pl.DeviceIdType.MESH,
                ).wait_send()
                pltpu.make_async_remote_copy(
                    src_ref=v_ref.at[:, :, d * H_LOC:(d + 1) * H_LOC, :],
                    dst_ref=v_buf,
                    send_sem=scat_send_sems.at[d - 1, 1],
                    recv_sem=kv_recv_sems.at[1],
                    device_id=(d,),
                    device_id_type=pl.DeviceIdType.MESH,
                ).wait_send()

        @pl.when(my_i != 0)
        def _():
            pltpu.make_async_remote_copy(
                src_ref=k_buf, dst_ref=k_buf,
                send_sem=scat_send_sems.at[0, 0],
                recv_sem=kv_recv_sems.at[0],
                device_id=(0,),
                device_id_type=pl.DeviceIdType.MESH,
            ).wait_recv()
            pltpu.make_async_remote_copy(
                src_ref=v_buf, dst_ref=v_buf,
                send_sem=scat_send_sems.at[0, 1],
                recv_sem=kv_recv_sems.at[1],
                device_id=(0,),
                device_id_type=pl.DeviceIdType.MESH,
            ).wait_recv()

        qb = lax.broadcasted_iota(jnp.int32, (SQ, SKV), 0) // BLK
        kb = lax.broadcasted_iota(jnp.int32, (SQ, SKV), 1) // BLK
        mask = kb <= qb

        for b in range(B):
            for h in range(H_LOC):
                q = q_buf[b, :, h * DH:(h + 1) * DH]
                k = k_buf[b, :, h, :]
                s = lax.dot_general(
                    q, k, (((1,), (1,)), ((), ())),
                    preferred_element_type=jnp.float32) * 0.125
                s = jnp.where(mask, s, -1e9)
                m = jnp.max(s, axis=1, keepdims=True)
                w = jnp.exp(s - m)
                w = w / jnp.sum(w, axis=1, keepdims=True)
                ctx_buf[b, :, h * DH:(h + 1) * DH] = jnp.dot(
                    w, v_buf[b, :, h, :],
                    preferred_element_type=jnp.float32)

        for b in range(B):
            part = jnp.dot(ctx_buf[b, :, :], wo_ref[:, :],
                           preferred_element_type=jnp.float32)
            out_ref[b, :, :] = part
            comm[0, b, :, :] = part

        for h in range(N_DEV - 1):
            rdma = pltpu.make_async_remote_copy(
                src_ref=comm.at[h],
                dst_ref=comm.at[h + 1],
                send_sem=ring_send_sems.at[h],
                recv_sem=ring_recv_sems.at[h],
                device_id=(right,),
                device_id_type=pl.DeviceIdType.MESH,
            )
            rdma.start()
            rdma.wait()
            out_ref[:, :, :] += comm[h + 1]

    return pl.pallas_call(
        body,
        out_shape=jax.ShapeDtypeStruct((B, SQ, D_MODEL), jnp.float32),
        in_specs=[
            pl.BlockSpec(memory_space=pltpu.VMEM),
            pl.BlockSpec(memory_space=pltpu.VMEM),
            pl.BlockSpec(memory_space=pl.ANY),
            pl.BlockSpec(memory_space=pl.ANY),
            pl.BlockSpec(memory_space=pltpu.VMEM),
        ],
        out_specs=pl.BlockSpec(memory_space=pltpu.VMEM),
        compiler_params=pltpu.CompilerParams(
            vmem_limit_bytes=100 * 1024 * 1024,
            collective_id=0,
        ),
        scratch_shapes=[
            pltpu.VMEM((B, SKV, H_LOC, DH), jnp.float32),
            pltpu.VMEM((B, SKV, H_LOC, DH), jnp.float32),
            pltpu.VMEM((B, SQ, H_LOC * DH), jnp.float32),
            pltpu.VMEM((B, SQ, H_LOC * DH), jnp.float32),
            pltpu.VMEM((N_DEV, B, SQ, D_MODEL), jnp.float32),
            pltpu.SemaphoreType.DMA((N_DEV - 1, 2)),
            pltpu.SemaphoreType.DMA((2,)),
            pltpu.SemaphoreType.DMA((N_DEV - 1,)),
            pltpu.SemaphoreType.DMA((N_DEV - 1,)),
            pltpu.SemaphoreType.DMA((2,)),
        ],
    )(x, Wq, K_ext, V_ext, Wo)


# device time: 556787 ns/iter; 1.2497x vs baseline; 1.2497x over previous
import jax
import jax.numpy as jnp
from jax import lax
from jax.experimental import pallas as pl
from jax.experimental.pallas import tpu as pltpu

N_DEV = 8
B = 2
SQ = 512
SKV = 512
H_LOC = 8
DH = 64
BLK = 64
D_MODEL = 768


def kernel(x, Wq, K_ext, V_ext, Wo):
    def body(x_ref, wq_ref, k_ref, v_ref, wo_ref, out_ref,
             k_buf, v_buf, q_buf, ctx_buf, comm,
             scat_send_sems, kv_recv_sems, ring_send_sems, ring_recv_sems,
             local_sems):
        my_i = lax.axis_index("i")
        right = lax.rem(my_i + 1, N_DEV)

        barrier_sem = pltpu.get_barrier_semaphore()
        for d in range(N_DEV):
            pl.semaphore_signal(
                barrier_sem, inc=1,
                device_id=(d,), device_id_type=pl.DeviceIdType.MESH)
        pl.semaphore_wait(barrier_sem, N_DEV)

        @pl.when(my_i == 0)
        def _():
            ck = pltpu.make_async_copy(
                k_ref.at[:, :, 0:H_LOC, :], k_buf, local_sems.at[0])
            cv = pltpu.make_async_copy(
                v_ref.at[:, :, 0:H_LOC, :], v_buf, local_sems.at[1])
            ck.start()
            cv.start()
            rdmas = []
            for d in range(1, N_DEV):
                rk = pltpu.make_async_remote_copy(
                    src_ref=k_ref.at[:, :, d * H_LOC:(d + 1) * H_LOC, :],
                    dst_ref=k_buf,
                    send_sem=scat_send_sems.at[d - 1, 0],
                    recv_sem=kv_recv_sems.at[0],
                    device_id=(d,),
                    device_id_type=pl.DeviceIdType.MESH,
                )
                rv = pltpu.make_async_remote_copy(
                    src_ref=v_ref.at[:, :, d * H_LOC:(d + 1) * H_LOC, :],
                    dst_ref=v_buf,
                    send_sem=scat_send_sems.at[d - 1, 1],
                    recv_sem=kv_recv_sems.at[1],
                    device_id=(d,),
                    device_id_type=pl.DeviceIdType.MESH,
                )
                rk.start()
                rv.start()
                rdmas.append((rk, rv))

        for b in range(B):
            q_buf[b, :, :] = jnp.dot(
                x_ref[b, :, :], wq_ref[:, :],
                preferred_element_type=jnp.float32)

        @pl.when(my_i == 0)
        def _():
            pltpu.make_async_copy(
                k_ref.at[:, :, 0:H_LOC, :], k_buf, local_sems.at[0]).wait()
            pltpu.make_async_copy(
                v_ref.at[:, :, 0:H_LOC, :], v_buf, local_sems.at[1]).wait()
            for d in range(1, N_DEV):
                pltpu.make_async_remote_copy(
                    src_ref=k_ref.at[:, :, d * H_LOC:(d + 1) * H_LOC, :],
                    dst_ref=k_buf,
                    send_sem=scat_send_sems.at[d - 1, 0],
                    recv_sem=kv_recv_sems.at[0],
                    device_id=(d,),
                    device_id_type=pl.DeviceIdType.MESH,
                ).wait_send()
                pltpu.make_async_remote_copy(
                    src_ref=v_ref.at[:, :, d * H_LOC:(d + 1) * H_LOC, :],
                    dst_ref=v_buf,
                    send_sem=scat_send_sems.at[d - 1, 1],
                    recv_sem=kv_recv_sems.at[1],
                    device_id=(d,),
                    device_id_type=pl.DeviceIdType.MESH,
                ).wait_send()

        @pl.when(my_i != 0)
        def _():
            pltpu.make_async_remote_copy(
                src_ref=k_buf, dst_ref=k_buf,
                send_sem=scat_send_sems.at[0, 0],
                recv_sem=kv_recv_sems.at[0],
                device_id=(0,),
                device_id_type=pl.DeviceIdType.MESH,
            ).wait_recv()
            pltpu.make_async_remote_copy(
                src_ref=v_buf, dst_ref=v_buf,
                send_sem=scat_send_sems.at[0, 1],
                recv_sem=kv_recv_sems.at[1],
                device_id=(0,),
                device_id_type=pl.DeviceIdType.MESH,
            ).wait_recv()

        qb = lax.broadcasted_iota(jnp.int32, (SQ, SKV), 0) // BLK
        kb = lax.broadcasted_iota(jnp.int32, (SQ, SKV), 1) // BLK
        mask = kb <= qb

        for b in range(B):
            for h in range(H_LOC):
                q = q_buf[b, :, h * DH:(h + 1) * DH]
                k = k_buf[b, :, h, :]
                s = lax.dot_general(
                    q, k, (((1,), (1,)), ((), ())),
                    preferred_element_type=jnp.float32) * 0.125
                s = jnp.where(mask, s, -1e9)
                m = jnp.max(s, axis=1, keepdims=True)
                w = jnp.exp(s - m)
                w = w / jnp.sum(w, axis=1, keepdims=True)
                ctx_buf[b, :, h * DH:(h + 1) * DH] = jnp.dot(
                    w, v_buf[b, :, h, :],
                    preferred_element_type=jnp.float32)

        for b in range(B):
            out_ref[b, :, :] = jnp.dot(ctx_buf[b, :, :], wo_ref[:, :],
                                       preferred_element_type=jnp.float32)

        for r, flip in enumerate((1, 3, 4)):
            partner = jnp.bitwise_xor(my_i, flip)
            rdma = pltpu.make_async_remote_copy(
                src_ref=out_ref,
                dst_ref=comm.at[r],
                send_sem=ring_send_sems.at[r],
                recv_sem=ring_recv_sems.at[r],
                device_id=(partner,),
                device_id_type=pl.DeviceIdType.MESH,
            )
            rdma.start()
            rdma.wait()
            out_ref[:, :, :] += comm[r]

    return pl.pallas_call(
        body,
        out_shape=jax.ShapeDtypeStruct((B, SQ, D_MODEL), jnp.float32),
        in_specs=[
            pl.BlockSpec(memory_space=pltpu.VMEM),
            pl.BlockSpec(memory_space=pltpu.VMEM),
            pl.BlockSpec(memory_space=pl.ANY),
            pl.BlockSpec(memory_space=pl.ANY),
            pl.BlockSpec(memory_space=pltpu.VMEM),
        ],
        out_specs=pl.BlockSpec(memory_space=pltpu.VMEM),
        compiler_params=pltpu.CompilerParams(
            vmem_limit_bytes=100 * 1024 * 1024,
            collective_id=0,
        ),
        scratch_shapes=[
            pltpu.VMEM((B, SKV, H_LOC, DH), jnp.float32),
            pltpu.VMEM((B, SKV, H_LOC, DH), jnp.float32),
            pltpu.VMEM((B, SQ, H_LOC * DH), jnp.float32),
            pltpu.VMEM((B, SQ, H_LOC * DH), jnp.float32),
            pltpu.VMEM((3, B, SQ, D_MODEL), jnp.float32),
            pltpu.SemaphoreType.DMA((N_DEV - 1, 2)),
            pltpu.SemaphoreType.DMA((2,)),
            pltpu.SemaphoreType.DMA((3,)),
            pltpu.SemaphoreType.DMA((3,)),
            pltpu.SemaphoreType.DMA((2,)),
        ],
    )(x, Wq, K_ext, V_ext, Wo)
